# baseline (speedup 1.0000x reference)
import functools

import jax
import jax.numpy as jnp
from jax import lax
from jax.experimental import pallas as pl
from jax.experimental.pallas import tpu as pltpu
from jax.experimental.pallas import tpu_sc as plsc

B = 1024
S = 200
DIM = 64

_info = plsc.get_sparse_core_info()
NC, NS, L = _info.num_cores, _info.num_subcores, _info.num_lanes
NW = NC * NS
ROWS_PER_W = B // NW
TOK_PER_W = ROWS_PER_W * S
CHUNK = 80
NCHUNKS = TOK_PER_W // CHUNK
VPR = DIM // 16
RING = 4
LEAD = 2
NIDV = TOK_PER_W // 16


def _make_kernel():
  mesh = plsc.VectorSubcoreMesh(core_axis_name="c", subcore_axis_name="s")

  rows_scratch = [pltpu.VMEM((CHUNK, DIM), jnp.float32) for _ in range(RING)]
  sem_scratch = [pltpu.SemaphoreType.DMA for _ in range(2 * RING)]

  @functools.partial(
      pl.kernel,
      mesh=mesh,
      compiler_params=pltpu.CompilerParams(
          use_tc_tiling_on_sc=False, needs_layout_passes=False),
      out_type=jax.ShapeDtypeStruct((B * S, DIM), jnp.float32),
      scratch_types=[
          pltpu.VMEM((S, DIM), jnp.float32),
          pltpu.VMEM((DIM, S), jnp.float32),
          pltpu.VMEM((S, ROWS_PER_W), jnp.float32),
          pltpu.VMEM((TOK_PER_W,), jnp.int32),
      ] + rows_scratch + sem_scratch,
  )
  def k(idsf_hbm, table_hbm, pos_hbm, out_hbm, pos_v, post_v, idsf_v, idx_v,
        *rest):
    bufs = rest[:RING]
    gsems = rest[RING:2 * RING]
    ssems = rest[2 * RING:]
    wid = lax.axis_index("s") * NC + lax.axis_index("c")
    pltpu.sync_copy(pos_hbm.at[:, pl.ds(0, S)], post_v)
    row0 = wid * ROWS_PER_W
    base0 = wid * TOK_PER_W
    pltpu.sync_copy(idsf_hbm.at[:, pl.ds(row0, ROWS_PER_W)], idsf_v)

    lanes = lax.iota(jnp.int32, 16)

    def cvt_body(v, carry):
      f = lanes + v * 16
      r = lax.div(f, S)
      p = f - r * S
      x = plsc.load_gather(idsf_v, [p, r])
      idx_v[pl.ds(v * 16, 16)] = (
          plsc.bitcast(x, jnp.int32) & jnp.int32(0x007FFFFF))
      return carry

    lax.fori_loop(0, NIDV, cvt_body, 0, unroll=8)

    def pos_body(pr, carry):
      prv = lanes * 0 + pr
      for kk in range(VPR):
        x = plsc.load_gather(post_v, [lanes + 16 * kk, prv])
        pos_v[pr, pl.ds(kk * 16, 16)] = x
      return carry

    lax.fori_loop(0, S, pos_body, 0, unroll=8)

    def issue_gather(c, b):
      pltpu.async_copy(
          table_hbm.at[idx_v.at[pl.ds(c * CHUNK, CHUNK)]], bufs[b], gsems[b])

    def wait_gather(b):
      pltpu.make_async_copy(
          table_hbm.at[pl.ds(0, CHUNK)], bufs[b], gsems[b]).wait()

    def wait_store(b):
      pltpu.make_async_copy(
          bufs[b], out_hbm.at[pl.ds(0, CHUNK)], ssems[b]).wait()

    for c in range(LEAD):
      issue_gather(c, c % RING)

    def step(c, b):
      @pl.when(c + LEAD < NCHUNKS)
      def _():
        bp = (b + LEAD) % RING

        @pl.when(c >= RING - LEAD)
        def _():
          wait_store(bp)

        issue_gather(c + LEAD, bp)

      wait_gather(b)
      cur = bufs[b]
      prow0 = lax.rem(c * CHUNK, S)

      def row_body(rr, carry2):
        pr = lax.rem(prow0 + rr, S)
        for kk in range(VPR):
          sl = pl.ds(kk * 16, 16)
          plsc.addupdate(cur.at[rr, sl], pos_v[pr, sl])
        return carry2

      lax.fori_loop(0, CHUNK, row_body, 0, unroll=8)
      pltpu.async_copy(
          cur, out_hbm.at[pl.ds(base0 + c * CHUNK, CHUNK)], ssems[b])

    def ring_body(j, carry):
      for b in range(RING):
        step(j * RING + b, b)
      return carry

    lax.fori_loop(0, NCHUNKS // RING, ring_body, 0)

    for b in range(RING):
      wait_store(b)

  return k


_kernel = _make_kernel()


def kernel(input_ids, token_table, position_embedding):
  Bq, Sq = input_ids.shape
  ids_tagged = input_ids.astype(jnp.int32) | jnp.int32(0x4B000000)
  ids_f = lax.bitcast_convert_type(ids_tagged, jnp.float32)
  out = _kernel(ids_f.T, token_table, position_embedding[0].T)
  return out.reshape(Bq, Sq, DIM)


# speedup vs baseline: 1.0260x; 1.0260x over previous
import functools

import jax
import jax.numpy as jnp
from jax import lax
from jax.experimental import pallas as pl
from jax.experimental.pallas import tpu as pltpu
from jax.experimental.pallas import tpu_sc as plsc

B = 1024
S = 200
DIM = 64
PADDIM = 128

_info = plsc.get_sparse_core_info()
NC, NS, L = _info.num_cores, _info.num_subcores, _info.num_lanes
NW = NC * NS
ROWS_PER_W = B // NW
TOK_PER_W = ROWS_PER_W * S
CHUNK = 80
NCHUNKS = TOK_PER_W // CHUNK
VPR = DIM // 16
RING = 4
LEAD = 2
NIDV = TOK_PER_W // 16


def _make_kernel():
  mesh = plsc.VectorSubcoreMesh(core_axis_name="c", subcore_axis_name="s")

  gbufs_s = [pltpu.VMEM((CHUNK, PADDIM), jnp.float32) for _ in range(RING)]
  sbufs_s = [pltpu.VMEM((CHUNK, DIM), jnp.float32) for _ in range(RING)]
  sem_s = [pltpu.SemaphoreType.DMA for _ in range(2 * RING)]

  @functools.partial(
      pl.kernel,
      mesh=mesh,
      compiler_params=pltpu.CompilerParams(
          use_tc_tiling_on_sc=False, needs_layout_passes=False),
      out_type=jax.ShapeDtypeStruct((B * S, DIM), jnp.float32),
      scratch_types=[
          pltpu.VMEM((S, DIM), jnp.float32),
          pltpu.VMEM((DIM, S), jnp.float32),
          pltpu.VMEM((S, ROWS_PER_W), jnp.float32),
          pltpu.VMEM((TOK_PER_W,), jnp.int32),
      ] + gbufs_s + sbufs_s + sem_s,
  )
  def k(idsf_hbm, table_hbm, pos_hbm, out_hbm, pos_v, post_v, idsf_v, idx_v,
        *rest):
    gbufs = rest[:RING]
    sbufs = rest[RING:2 * RING]
    gsems = rest[2 * RING:3 * RING]
    ssems = rest[3 * RING:]
    wid = lax.axis_index("s") * NC + lax.axis_index("c")
    pltpu.sync_copy(pos_hbm.at[:, pl.ds(0, S)], post_v)
    row0 = wid * ROWS_PER_W
    base0 = wid * TOK_PER_W
    pltpu.sync_copy(idsf_hbm.at[:, pl.ds(row0, ROWS_PER_W)], idsf_v)

    lanes = lax.iota(jnp.int32, 16)

    def cvt_body(v, carry):
      f = lanes + v * 16
      r = lax.div(f, S)
      p = f - r * S
      x = plsc.load_gather(idsf_v, [p, r])
      idx_v[pl.ds(v * 16, 16)] = (
          plsc.bitcast(x, jnp.int32) & jnp.int32(0x007FFFFF))
      return carry

    lax.fori_loop(0, NIDV, cvt_body, 0, unroll=8)

    def pos_body(pr, carry):
      prv = lanes * 0 + pr
      for kk in range(VPR):
        x = plsc.load_gather(post_v, [lanes + 16 * kk, prv])
        pos_v[pr, pl.ds(kk * 16, 16)] = x
      return carry

    lax.fori_loop(0, S, pos_body, 0, unroll=8)

    def issue_gather(c, b):
      pltpu.async_copy(
          table_hbm.at[idx_v.at[pl.ds(c * CHUNK, CHUNK)]], gbufs[b], gsems[b])

    def wait_gather(b):
      pltpu.make_async_copy(
          table_hbm.at[pl.ds(0, CHUNK)], gbufs[b], gsems[b]).wait()

    def wait_store(b):
      pltpu.make_async_copy(
          sbufs[b], out_hbm.at[pl.ds(0, CHUNK)], ssems[b]).wait()

    for c in range(LEAD):
      issue_gather(c, c % RING)

    def step(c, b):
      @pl.when(c + LEAD < NCHUNKS)
      def _():
        bp = (b + LEAD) % RING

        @pl.when(c >= RING - LEAD)
        def _():
          wait_store(bp)

        issue_gather(c + LEAD, bp)

      wait_gather(b)
      gcur = gbufs[b]
      scur = sbufs[b]
      prow0 = lax.rem(c * CHUNK, S)

      def row_body(rr, carry2):
        pr = lax.rem(prow0 + rr, S)
        for kk in range(VPR):
          sl = pl.ds(kk * 16, 16)
          scur[rr, sl] = gcur[rr, sl] + pos_v[pr, sl]
        return carry2

      lax.fori_loop(0, CHUNK, row_body, 0, unroll=8)
      pltpu.async_copy(
          scur, out_hbm.at[pl.ds(base0 + c * CHUNK, CHUNK)], ssems[b])

    def ring_body(j, carry):
      for b in range(RING):
        step(j * RING + b, b)
      return carry

    lax.fori_loop(0, NCHUNKS // RING, ring_body, 0)

    for b in range(RING):
      wait_store(b)

  return k


_kernel = _make_kernel()


def kernel(input_ids, token_table, position_embedding):
  Bq, Sq = input_ids.shape
  ids_tagged = input_ids.astype(jnp.int32) | jnp.int32(0x4B000000)
  ids_f = lax.bitcast_convert_type(ids_tagged, jnp.float32)
  table_p = jnp.pad(token_table, ((0, 0), (0, PADDIM - DIM)))
  out = _kernel(ids_f.T, table_p, position_embedding[0].T)
  return out.reshape(Bq, Sq, DIM)


# speedup vs baseline: 1.0786x; 1.0512x over previous
import functools

import jax
import jax.numpy as jnp
from jax import lax
from jax.experimental import pallas as pl
from jax.experimental.pallas import tpu as pltpu
from jax.experimental.pallas import tpu_sc as plsc

B = 1024
S = 200
DIM = 64
PADDIM = 128
VOCAB_ROWS = 1000000

_info = plsc.get_sparse_core_info()
NC, NS, L = _info.num_cores, _info.num_subcores, _info.num_lanes
NW = NC * NS
ROWS_PER_W = B // NW
TOK_PER_W = ROWS_PER_W * S
CHUNK = 80
NCHUNKS = TOK_PER_W // CHUNK
VPR = DIM // 16
RING = 4
LEAD = 2
NIDV = TOK_PER_W // 16


def _make_kernel():
  mesh = plsc.VectorSubcoreMesh(core_axis_name="c", subcore_axis_name="s")

  gbufs_s = [pltpu.VMEM((CHUNK, DIM), jnp.float32) for _ in range(RING)]
  sem_s = [pltpu.SemaphoreType.DMA for _ in range(2 * RING)]

  @functools.partial(
      pl.kernel,
      mesh=mesh,
      compiler_params=pltpu.CompilerParams(
          use_tc_tiling_on_sc=False, needs_layout_passes=False),
      out_type=jax.ShapeDtypeStruct((B * S, DIM), jnp.float32),
      scratch_types=[
          pltpu.VMEM((S, DIM), jnp.float32),
          pltpu.VMEM((DIM, S), jnp.float32),
          pltpu.VMEM((S, ROWS_PER_W), jnp.float32),
          pltpu.VMEM((TOK_PER_W,), jnp.int32),
      ] + gbufs_s + sem_s,
  )
  def k(idsf_hbm, table_hbm, pos_hbm, out_hbm, pos_v, post_v, idsf_v, idx_v,
        *rest):
    gbufs = rest[:RING]
    gsems = rest[RING:2 * RING]
    ssems = rest[2 * RING:]
    wid = lax.axis_index("s") * NC + lax.axis_index("c")
    pltpu.sync_copy(pos_hbm.at[:, pl.ds(0, S)], post_v)
    row0 = wid * ROWS_PER_W
    base0 = wid * TOK_PER_W
    pltpu.sync_copy(idsf_hbm.at[:, pl.ds(row0, ROWS_PER_W)], idsf_v)

    lanes = lax.iota(jnp.int32, 16)

    def cvt_body(v, carry):
      f = lanes + v * 16
      r = lax.div(f, S)
      p = f - r * S
      x = plsc.load_gather(idsf_v, [p, r])
      idx_v[pl.ds(v * 16, 16)] = (
          plsc.bitcast(x, jnp.int32) & jnp.int32(0x007FFFFF)) * 2
      return carry

    lax.fori_loop(0, NIDV, cvt_body, 0, unroll=8)

    def pos_body(pr, carry):
      prv = lanes * 0 + pr
      for kk in range(VPR):
        x = plsc.load_gather(post_v, [lanes + 16 * kk, prv])
        pos_v[pr, pl.ds(kk * 16, 16)] = x
      return carry

    lax.fori_loop(0, S, pos_body, 0, unroll=8)

    def issue_gather(c, b):
      pltpu.async_copy(
          table_hbm.at[idx_v.at[pl.ds(c * CHUNK, CHUNK)]], gbufs[b], gsems[b])

    def wait_gather(b):
      pltpu.make_async_copy(
          table_hbm.at[pl.ds(0, CHUNK)], gbufs[b], gsems[b]).wait()

    def wait_store(b):
      pltpu.make_async_copy(
          gbufs[b], out_hbm.at[pl.ds(0, CHUNK)], ssems[b]).wait()

    for c in range(LEAD):
      issue_gather(c, c % RING)

    def step(c, b):
      @pl.when(c + LEAD < NCHUNKS)
      def _():
        bp = (b + LEAD) % RING

        @pl.when(c >= RING - LEAD)
        def _():
          wait_store(bp)

        issue_gather(c + LEAD, bp)

      wait_gather(b)
      gcur = gbufs[b]
      prow0 = lax.rem(c * CHUNK, S)

      def row_body(rr, carry2):
        pr = lax.rem(prow0 + rr, S)
        for kk in range(VPR):
          sl = pl.ds(kk * 16, 16)
          plsc.addupdate(gcur.at[rr, sl], pos_v[pr, sl])
        return carry2

      lax.fori_loop(0, CHUNK, row_body, 0, unroll=8)
      pltpu.async_copy(
          gcur, out_hbm.at[pl.ds(base0 + c * CHUNK, CHUNK)], ssems[b])

    def ring_body(j, carry):
      for b in range(RING):
        step(j * RING + b, b)
      return carry

    lax.fori_loop(0, NCHUNKS // RING, ring_body, 0)

    for b in range(RING):
      wait_store(b)

  return k


_kernel = _make_kernel()


def kernel(input_ids, token_table, position_embedding):
  Bq, Sq = input_ids.shape
  ids_tagged = input_ids.astype(jnp.int32) | jnp.int32(0x4B000000)
  ids_f = lax.bitcast_convert_type(ids_tagged, jnp.float32)
  table_p = jnp.pad(token_table, ((0, 0), (0, PADDIM - DIM)))
  table_p = table_p.reshape(2 * VOCAB_ROWS, DIM)
  out = _kernel(ids_f.T, table_p, position_embedding[0].T)
  return out.reshape(Bq, Sq, DIM)
